# baseline (speedup 1.0000x reference)
import functools

import jax
import jax.numpy as jnp
from jax import lax
from jax.experimental import pallas as pl
from jax.experimental.pallas import tpu as pltpu
from jax.experimental.pallas import tpu_sc as plsc

N_NODES = 512
KSIZE = 16
N_CHAN = 8
N_STEPS = 2
NUM_WORKERS = 32
ROWS_PER_W = N_NODES // NUM_WORKERS


def _worker_base():
    wid = lax.axis_index("s") * 2 + lax.axis_index("c")
    return wid * ROWS_PER_W


def _masked_rowsums(m_v, x_v):
    def body(t, accs):
        xs = [x_v[t * 16 + l, :] for l in range(16)]
        out = []
        for r in range(ROWS_PER_W):
            mv = m_v[r, pl.ds(t * 16, 16)]
            mf = jnp.minimum(mv, 1).astype(jnp.float32)
            terms = [xs[l] * mf[l] for l in range(16)]
            while len(terms) > 1:
                terms = [terms[i] + terms[i + 1]
                         for i in range(0, len(terms), 2)]
            out.append(accs[r] + terms[0])
        return tuple(out)

    zero = jnp.zeros((KSIZE,), jnp.float32)
    return lax.fori_loop(0, N_NODES // 16, body,
                         tuple(zero for _ in range(ROWS_PER_W)))


@functools.cache
def _build_calls():
    mesh = plsc.VectorSubcoreMesh(core_axis_name="c", subcore_axis_name="s")

    @functools.partial(
        pl.kernel,
        out_type=jax.ShapeDtypeStruct((N_NODES, KSIZE), jnp.float32),
        mesh=mesh,
        scratch_types=[
            pltpu.VMEM((ROWS_PER_W, N_NODES), jnp.int32),
            pltpu.VMEM((N_NODES, KSIZE), jnp.float32),
            pltpu.VMEM((ROWS_PER_W, KSIZE), jnp.float32),
        ],
    )
    def aggregate(m_hbm, x_hbm, out_hbm, m_v, x_v, o_v):
        base = _worker_base()
        pltpu.sync_copy(m_hbm.at[pl.ds(base, ROWS_PER_W), :], m_v)
        pltpu.sync_copy(x_hbm, x_v)
        rows = _masked_rowsums(m_v, x_v)
        for r in range(ROWS_PER_W):
            o_v[r, :] = rows[r]
        pltpu.sync_copy(o_v, out_hbm.at[pl.ds(base, ROWS_PER_W), :])

    @functools.partial(
        pl.kernel,
        out_type=jax.ShapeDtypeStruct((N_CHAN * N_NODES, KSIZE), jnp.float32),
        mesh=mesh,
        scratch_types=[
            pltpu.VMEM((ROWS_PER_W, N_NODES), jnp.int32),
            pltpu.VMEM((N_NODES, KSIZE), jnp.float32),
            pltpu.VMEM((ROWS_PER_W, KSIZE), jnp.float32),
            pltpu.VMEM((N_CHAN * N_STEPS, KSIZE), jnp.float32),
            pltpu.VMEM((N_CHAN, ROWS_PER_W, KSIZE), jnp.float32),
        ],
    )
    def aggregate_combine(m_hbm, p_hbm, l_hbm, k_hbm, out_hbm,
                          m_v, p_v, l_v, k_v, o_v):
        base = _worker_base()
        pltpu.sync_copy(m_hbm.at[pl.ds(base, ROWS_PER_W), :], m_v)
        pltpu.sync_copy(p_hbm, p_v)
        pltpu.sync_copy(l_hbm.at[pl.ds(base, ROWS_PER_W), :], l_v)
        pltpu.sync_copy(k_hbm, k_v)
        qs = _masked_rowsums(m_v, p_v)
        for r in range(ROWS_PER_W):
            q = qs[r]
            p_i = p_v[base + r, :]
            l_i = l_v[r, :]
            for c in range(N_CHAN):
                k0 = k_v[2 * c, :]
                k1 = k_v[2 * c + 1, :]
                o_v[c, r, :] = l_i + p_i * (k0 + k1) + q * (k0 * k1)
        for c in range(N_CHAN):
            pltpu.sync_copy(
                o_v.at[c],
                out_hbm.at[pl.ds(c * N_NODES + base, ROWS_PER_W), :])

    return aggregate, aggregate_combine


def kernel(labelsList, ligand_structure, kernels):
    aggregate, aggregate_combine = _build_calls()
    p = aggregate(ligand_structure, labelsList)
    flat_k = kernels.reshape(N_CHAN * N_STEPS, KSIZE)
    out = aggregate_combine(ligand_structure, p, labelsList, flat_k)
    return out.reshape(N_CHAN, N_NODES, KSIZE)


# speedup vs baseline: 1.5313x; 1.5313x over previous
import functools

import jax
import jax.numpy as jnp
from jax import lax
from jax.experimental import pallas as pl
from jax.experimental.pallas import tpu as pltpu
from jax.experimental.pallas import tpu_sc as plsc

N_NODES = 512
KSIZE = 16
N_CHAN = 8
N_STEPS = 2
NUM_WORKERS = 32
ROWS_PER_W = N_NODES // NUM_WORKERS


def _worker_base():
    wid = lax.axis_index("s") * 2 + lax.axis_index("c")
    return wid * ROWS_PER_W


def _masked_rowsums(m_v, x_v):
    group = 4

    def make_body(r0):
        def body(t, accs):
            xs = [x_v[t * 16 + l, :] for l in range(16)]
            out = []
            for g in range(group):
                mv = m_v[r0 + g, pl.ds(t * 16, 16)]
                mf = jnp.minimum(mv, 1).astype(jnp.float32)
                terms = [xs[l] * mf[l] for l in range(16)]
                while len(terms) > 1:
                    terms = [terms[i] + terms[i + 1]
                             for i in range(0, len(terms), 2)]
                out.append(accs[g] + terms[0])
            return tuple(out)

        return body

    zero = jnp.zeros((KSIZE,), jnp.float32)
    rows = []
    for r0 in range(0, ROWS_PER_W, group):
        accs = lax.fori_loop(0, N_NODES // 16, make_body(r0),
                             tuple(zero for _ in range(group)))
        rows.extend(accs)
    return rows


@functools.cache
def _build_calls():
    mesh = plsc.VectorSubcoreMesh(core_axis_name="c", subcore_axis_name="s")

    @functools.partial(
        pl.kernel,
        out_type=jax.ShapeDtypeStruct((N_NODES, KSIZE), jnp.float32),
        mesh=mesh,
        scratch_types=[
            pltpu.VMEM((ROWS_PER_W, N_NODES), jnp.int32),
            pltpu.VMEM((N_NODES, KSIZE), jnp.float32),
            pltpu.VMEM((ROWS_PER_W, KSIZE), jnp.float32),
        ],
    )
    def aggregate(m_hbm, x_hbm, out_hbm, m_v, x_v, o_v):
        base = _worker_base()
        pltpu.sync_copy(m_hbm.at[pl.ds(base, ROWS_PER_W), :], m_v)
        pltpu.sync_copy(x_hbm, x_v)
        rows = _masked_rowsums(m_v, x_v)
        for r in range(ROWS_PER_W):
            o_v[r, :] = rows[r]
        pltpu.sync_copy(o_v, out_hbm.at[pl.ds(base, ROWS_PER_W), :])

    @functools.partial(
        pl.kernel,
        out_type=jax.ShapeDtypeStruct((N_CHAN * N_NODES, KSIZE), jnp.float32),
        mesh=mesh,
        scratch_types=[
            pltpu.VMEM((ROWS_PER_W, N_NODES), jnp.int32),
            pltpu.VMEM((N_NODES, KSIZE), jnp.float32),
            pltpu.VMEM((ROWS_PER_W, KSIZE), jnp.float32),
            pltpu.VMEM((N_CHAN * N_STEPS, KSIZE), jnp.float32),
            pltpu.VMEM((N_CHAN, ROWS_PER_W, KSIZE), jnp.float32),
        ],
    )
    def aggregate_combine(m_hbm, p_hbm, l_hbm, k_hbm, out_hbm,
                          m_v, p_v, l_v, k_v, o_v):
        base = _worker_base()
        pltpu.sync_copy(m_hbm.at[pl.ds(base, ROWS_PER_W), :], m_v)
        pltpu.sync_copy(p_hbm, p_v)
        pltpu.sync_copy(l_hbm.at[pl.ds(base, ROWS_PER_W), :], l_v)
        pltpu.sync_copy(k_hbm, k_v)
        qs = _masked_rowsums(m_v, p_v)
        for r in range(ROWS_PER_W):
            q = qs[r]
            p_i = p_v[base + r, :]
            l_i = l_v[r, :]
            for c in range(N_CHAN):
                k0 = k_v[2 * c, :]
                k1 = k_v[2 * c + 1, :]
                o_v[c, r, :] = l_i + p_i * (k0 + k1) + q * (k0 * k1)
        for c in range(N_CHAN):
            pltpu.sync_copy(
                o_v.at[c],
                out_hbm.at[pl.ds(c * N_NODES + base, ROWS_PER_W), :])

    return aggregate, aggregate_combine


def kernel(labelsList, ligand_structure, kernels):
    aggregate, aggregate_combine = _build_calls()
    p = aggregate(ligand_structure, labelsList)
    flat_k = kernels.reshape(N_CHAN * N_STEPS, KSIZE)
    out = aggregate_combine(ligand_structure, p, labelsList, flat_k)
    return out.reshape(N_CHAN, N_NODES, KSIZE)


# speedup vs baseline: 1.7120x; 1.1180x over previous
import functools

import jax
import jax.numpy as jnp
from jax import lax
from jax.experimental import pallas as pl
from jax.experimental.pallas import tpu as pltpu
from jax.experimental.pallas import tpu_sc as plsc

N_NODES = 512
KSIZE = 16
N_CHAN = 8
N_STEPS = 2
NUM_WORKERS = 32
ROWS_PER_W = N_NODES // NUM_WORKERS
N_GROUPS = N_NODES // 4


def _worker_base():
    wid = lax.axis_index("s") * 2 + lax.axis_index("c")
    return wid * ROWS_PER_W


def _build_tables(mt_v, x_v, tab_v, nib_v):
    def body(g, carry):
        ms = [jnp.minimum(mt_v[4 * g + k, :], 1) for k in range(4)]
        nib = ms[0] + (ms[1] << 1) + (ms[2] << 2) + (ms[3] << 3)
        nib_v[g, :] = (g << 8) + (nib << 4)
        xs = [x_v[4 * g + k, :] for k in range(4)]
        tab_v[pl.ds(g * 256, 16)] = jnp.zeros((KSIZE,), jnp.float32)
        vals = {}
        for s in range(1, 16):
            k = (s & -s).bit_length() - 1
            prev = s ^ (1 << k)
            vals[s] = xs[k] if prev == 0 else vals[prev] + xs[k]
            tab_v[pl.ds(g * 256 + s * 16, 16)] = vals[s]
        return carry

    lax.fori_loop(0, N_GROUPS, body, 0)


def _gather_accumulate(tab_v, nib_v, tr_v):
    def body(g, accs):
        base = nib_v[g, :]
        return tuple(accs[d] + plsc.load_gather(tab_v, [base + d])
                     for d in range(KSIZE))

    zero = jnp.zeros((KSIZE,), jnp.float32)
    accs = lax.fori_loop(0, N_GROUPS, body,
                         tuple(zero for _ in range(KSIZE)))
    for d in range(KSIZE):
        tr_v[pl.ds(d * 16, 16)] = accs[d]
    lanes16 = lax.iota(jnp.int32, 16) * 16
    return [plsc.load_gather(tr_v, [lanes16 + r]) for r in range(ROWS_PER_W)]


_SCRATCH_COMMON = [
    pltpu.VMEM((N_NODES, ROWS_PER_W), jnp.int32),
    pltpu.VMEM((N_NODES, KSIZE), jnp.float32),
    pltpu.VMEM((N_GROUPS * 256,), jnp.float32),
    pltpu.VMEM((N_GROUPS, 16), jnp.int32),
    pltpu.VMEM((256,), jnp.float32),
]


@functools.cache
def _build_calls():
    mesh = plsc.VectorSubcoreMesh(core_axis_name="c", subcore_axis_name="s")

    @functools.partial(
        pl.kernel,
        out_type=jax.ShapeDtypeStruct((N_NODES, KSIZE), jnp.float32),
        mesh=mesh,
        compiler_params=pltpu.CompilerParams(use_tc_tiling_on_sc=False, needs_layout_passes=False),
        scratch_types=_SCRATCH_COMMON + [
            pltpu.VMEM((ROWS_PER_W, KSIZE), jnp.float32),
        ],
    )
    def aggregate(mt_hbm, x_hbm, out_hbm, mt_v, x_v, tab_v, nib_v, tr_v, o_v):
        base = _worker_base()
        pltpu.sync_copy(mt_hbm.at[:, pl.ds(base, ROWS_PER_W)], mt_v)
        pltpu.sync_copy(x_hbm, x_v)
        _build_tables(mt_v, x_v, tab_v, nib_v)
        rows = _gather_accumulate(tab_v, nib_v, tr_v)
        for r in range(ROWS_PER_W):
            o_v[r, :] = rows[r]
        pltpu.sync_copy(o_v, out_hbm.at[pl.ds(base, ROWS_PER_W), :])

    @functools.partial(
        pl.kernel,
        out_type=jax.ShapeDtypeStruct((N_CHAN * N_NODES, KSIZE), jnp.float32),
        mesh=mesh,
        compiler_params=pltpu.CompilerParams(use_tc_tiling_on_sc=False, needs_layout_passes=False),
        scratch_types=_SCRATCH_COMMON + [
            pltpu.VMEM((ROWS_PER_W, KSIZE), jnp.float32),
            pltpu.VMEM((N_CHAN * N_STEPS, KSIZE), jnp.float32),
            pltpu.VMEM((N_CHAN, ROWS_PER_W, KSIZE), jnp.float32),
        ],
    )
    def aggregate_combine(mt_hbm, p_hbm, l_hbm, k_hbm, out_hbm,
                          mt_v, p_v, tab_v, nib_v, tr_v, l_v, k_v, o_v):
        base = _worker_base()
        pltpu.sync_copy(mt_hbm.at[:, pl.ds(base, ROWS_PER_W)], mt_v)
        pltpu.sync_copy(p_hbm, p_v)
        pltpu.sync_copy(l_hbm.at[pl.ds(base, ROWS_PER_W), :], l_v)
        pltpu.sync_copy(k_hbm, k_v)
        _build_tables(mt_v, p_v, tab_v, nib_v)
        qs = _gather_accumulate(tab_v, nib_v, tr_v)
        for r in range(ROWS_PER_W):
            q = qs[r]
            p_i = p_v[base + r, :]
            l_i = l_v[r, :]
            for c in range(N_CHAN):
                k0 = k_v[2 * c, :]
                k1 = k_v[2 * c + 1, :]
                o_v[c, r, :] = l_i + p_i * (k0 + k1) + q * (k0 * k1)
        for c in range(N_CHAN):
            pltpu.sync_copy(
                o_v.at[c],
                out_hbm.at[pl.ds(c * N_NODES + base, ROWS_PER_W), :])

    return aggregate, aggregate_combine


def kernel(labelsList, ligand_structure, kernels):
    aggregate, aggregate_combine = _build_calls()
    mt = ligand_structure.T
    p = aggregate(mt, labelsList)
    flat_k = kernels.reshape(N_CHAN * N_STEPS, KSIZE)
    out = aggregate_combine(mt, p, labelsList, flat_k)
    return out.reshape(N_CHAN, N_NODES, KSIZE)
